# baseline (device time: 210102 ns/iter reference)
import functools

import numpy as np

import jax
import jax.numpy as jnp
from jax import lax
from jax.experimental import pallas as pl
from jax.experimental.pallas import tpu as pltpu

N_DEV = 16
B, SQ, SKV, HQ_TOTAL, DH = 2, 512, 512, 128, 64
H_PER = HQ_TOTAL // N_DEV
D_MODEL = 768
ROWS = B * SQ
CHUNK = ROWS // N_DEV

_qi = np.arange(SQ)[:, None]
_ki = np.arange(SKV)[None, :]
_MASK = (np.abs(_qi - _ki) <= 128) | (_ki < 32) | (_qi < 32)


def _ring_allreduce(partial):

    def body(p_ref, out_ref, acc_ref, rbuf_ref,
             rs_send, rs_recv, ag_send, ag_recv):
        my = lax.axis_index("i")
        left = lax.rem(my + N_DEV - 1, N_DEV)
        right = lax.rem(my + 1, N_DEV)

        barrier_sem = pltpu.get_barrier_semaphore()
        for nbr in (left, right):
            pl.semaphore_signal(
                barrier_sem, inc=1,
                device_id=(nbr,), device_id_type=pl.DeviceIdType.MESH,
            )
        pl.semaphore_wait(barrier_sem, 2)

        for j in range(N_DEV):
            src_row = lax.rem(my + j, N_DEV) * CHUNK
            acc_ref[j, :, :] = p_ref[pl.ds(src_row, CHUNK), :]

        for s in range(N_DEV - 1):
            j_send = (N_DEV - s) % N_DEV
            rdma = pltpu.make_async_remote_copy(
                src_ref=acc_ref.at[j_send],
                dst_ref=rbuf_ref.at[s],
                send_sem=rs_send.at[s],
                recv_sem=rs_recv.at[s],
                device_id=(right,),
                device_id_type=pl.DeviceIdType.MESH,
            )
            rdma.start()
            rdma.wait()
            j_acc = N_DEV - 1 - s
            acc_ref[j_acc, :, :] = acc_ref[j_acc, :, :] + rbuf_ref[s, :, :]

        for s in range(N_DEV - 1):
            j_send = (1 - s) % N_DEV
            j_dst = (N_DEV - s) % N_DEV
            rdma = pltpu.make_async_remote_copy(
                src_ref=acc_ref.at[j_send],
                dst_ref=acc_ref.at[j_dst],
                send_sem=ag_send.at[s],
                recv_sem=ag_recv.at[s],
                device_id=(right,),
                device_id_type=pl.DeviceIdType.MESH,
            )
            rdma.start()
            rdma.wait()

        for j in range(N_DEV):
            dst_row = lax.rem(my + j, N_DEV) * CHUNK
            out_ref[pl.ds(dst_row, CHUNK), :] = acc_ref[j, :, :]

    return pl.pallas_call(
        body,
        out_shape=jax.ShapeDtypeStruct((ROWS, D_MODEL), jnp.float32),
        in_specs=[pl.BlockSpec(memory_space=pltpu.VMEM)],
        out_specs=pl.BlockSpec(memory_space=pltpu.VMEM),
        scratch_shapes=[
            pltpu.VMEM((N_DEV, CHUNK, D_MODEL), jnp.float32),
            pltpu.VMEM((N_DEV - 1, CHUNK, D_MODEL), jnp.float32),
            pltpu.SemaphoreType.DMA((N_DEV - 1,)),
            pltpu.SemaphoreType.DMA((N_DEV - 1,)),
            pltpu.SemaphoreType.DMA((N_DEV - 1,)),
            pltpu.SemaphoreType.DMA((N_DEV - 1,)),
        ],
        compiler_params=pltpu.CompilerParams(collective_id=0),
    )(partial)


def kernel(x, Wq, K_ext, V_ext, Wo):
    my = lax.axis_index("i")

    xb = x.astype(jnp.bfloat16)
    Q = jnp.einsum(
        "bsd,df->bsf", xb, Wq.astype(jnp.bfloat16),
        preferred_element_type=jnp.float32,
    ).reshape(B, SQ, H_PER, DH).astype(jnp.bfloat16)

    K = lax.dynamic_slice_in_dim(K_ext, my * H_PER, H_PER, axis=2)
    V = lax.dynamic_slice_in_dim(V_ext, my * H_PER, H_PER, axis=2)
    K = K.astype(jnp.bfloat16)
    V = V.astype(jnp.bfloat16)

    scores = jnp.einsum(
        "bihd,bjhd->bhij", Q, K, preferred_element_type=jnp.float32
    ) * 0.125
    scores = jnp.where(jnp.asarray(_MASK)[None, None, :, :], scores, -1e9)
    w = jax.nn.softmax(scores, axis=-1).astype(jnp.bfloat16)

    ctx = jnp.einsum(
        "bhij,bjhd->bihd", w, V, preferred_element_type=jnp.float32
    ).reshape(B, SQ, H_PER * DH).astype(jnp.bfloat16)

    partial = jnp.einsum(
        "bsf,fd->bsd", ctx, Wo.astype(jnp.bfloat16),
        preferred_element_type=jnp.float32,
    ).reshape(ROWS, D_MODEL)

    out = _ring_allreduce(partial)
    return out.reshape(B, SQ, D_MODEL)


# device time: 137847 ns/iter; 1.5242x vs baseline; 1.5242x over previous
import numpy as np

import jax
import jax.numpy as jnp
from jax import lax
from jax.experimental import pallas as pl
from jax.experimental.pallas import tpu as pltpu

N_DEV = 16
B, SQ, SKV, HQ_TOTAL, DH = 2, 512, 512, 128, 64
H_PER = HQ_TOTAL // N_DEV
D_MODEL = 768
ROWS = B * SQ

_HALVES = [512, 256, 128, 64]
_RBUF_OFFS = [0, 512, 768, 896]

_qi = np.arange(SQ)[:, None]
_ki = np.arange(SKV)[None, :]
_MASK = (np.abs(_qi - _ki) <= 128) | (_ki < 32) | (_qi < 32)


def _butterfly_allreduce(partial):

    def body(p_ref, out_ref, rbuf_ref, rs_send, rs_recv, ag_send, ag_recv):
        my = lax.axis_index("i")
        j = lax.rem(my, 4)
        z = lax.div(my, 4)
        bits = [
            jnp.where((j == 1) | (j == 2), 1, 0),
            jnp.where(j >= 2, 1, 0),
            lax.rem(z, 2),
            lax.div(z, 2),
        ]
        partners = [my ^ 1, my ^ 3, my ^ 4, my ^ 8]

        barrier_sem = pltpu.get_barrier_semaphore()
        for p in partners:
            pl.semaphore_signal(
                barrier_sem, inc=1,
                device_id=(p,), device_id_type=pl.DeviceIdType.MESH,
            )
        pl.semaphore_wait(barrier_sem, len(partners))

        out_ref[:, :] = p_ref[:, :]

        S = jnp.int32(0)
        for k in range(4):
            h = _HALVES[k]
            send_off = S + (1 - bits[k]) * h
            keep_off = S + bits[k] * h
            rdma = pltpu.make_async_remote_copy(
                src_ref=out_ref.at[pl.ds(send_off, h)],
                dst_ref=rbuf_ref.at[pl.ds(_RBUF_OFFS[k], h)],
                send_sem=rs_send.at[k],
                recv_sem=rs_recv.at[k],
                device_id=(partners[k],),
                device_id_type=pl.DeviceIdType.MESH,
            )
            rdma.start()
            rdma.wait()
            out_ref[pl.ds(keep_off, h), :] = (
                out_ref[pl.ds(keep_off, h), :]
                + rbuf_ref[pl.ds(_RBUF_OFFS[k], h), :]
            )
            S = keep_off

        for k in (3, 2, 1, 0):
            g = _HALVES[k]
            rdma = pltpu.make_async_remote_copy(
                src_ref=out_ref.at[pl.ds(S, g)],
                dst_ref=out_ref.at[pl.ds(S, g)],
                send_sem=ag_send.at[k],
                recv_sem=ag_recv.at[k],
                device_id=(partners[k],),
                device_id_type=pl.DeviceIdType.MESH,
            )
            rdma.start()
            rdma.wait()
            S = S - bits[k] * g

    return pl.pallas_call(
        body,
        out_shape=jax.ShapeDtypeStruct((ROWS, D_MODEL), jnp.bfloat16),
        in_specs=[pl.BlockSpec(memory_space=pltpu.VMEM)],
        out_specs=pl.BlockSpec(memory_space=pltpu.VMEM),
        scratch_shapes=[
            pltpu.VMEM((ROWS, D_MODEL), jnp.bfloat16),
            pltpu.SemaphoreType.DMA((4,)),
            pltpu.SemaphoreType.DMA((4,)),
            pltpu.SemaphoreType.DMA((4,)),
            pltpu.SemaphoreType.DMA((4,)),
        ],
        compiler_params=pltpu.CompilerParams(collective_id=0),
    )(partial)


def kernel(x, Wq, K_ext, V_ext, Wo):
    my = lax.axis_index("i")

    xb = x.astype(jnp.bfloat16)
    Q = jnp.einsum(
        "bsd,df->bsf", xb, Wq.astype(jnp.bfloat16),
        preferred_element_type=jnp.float32,
    ).reshape(B, SQ, H_PER, DH).astype(jnp.bfloat16)

    K = lax.dynamic_slice_in_dim(K_ext, my * H_PER, H_PER, axis=2)
    V = lax.dynamic_slice_in_dim(V_ext, my * H_PER, H_PER, axis=2)
    K = K.astype(jnp.bfloat16)
    V = V.astype(jnp.bfloat16)

    scores = jnp.einsum(
        "bihd,bjhd->bhij", Q, K, preferred_element_type=jnp.float32
    ) * 0.125
    scores = jnp.where(jnp.asarray(_MASK)[None, None, :, :], scores, -1e9)
    w = jax.nn.softmax(scores, axis=-1).astype(jnp.bfloat16)

    ctx = jnp.einsum(
        "bhij,bjhd->bihd", w, V, preferred_element_type=jnp.float32
    ).reshape(B, SQ, H_PER * DH).astype(jnp.bfloat16)

    partial = jnp.einsum(
        "bsf,fd->bsd", ctx, Wo.astype(jnp.bfloat16),
        preferred_element_type=jnp.float32,
    ).reshape(ROWS, D_MODEL).astype(jnp.bfloat16)

    out = _butterfly_allreduce(partial)
    return out.reshape(B, SQ, D_MODEL)


# device time: 119071 ns/iter; 1.7645x vs baseline; 1.1577x over previous
import jax
import jax.numpy as jnp
from jax import lax
from jax.experimental import pallas as pl
from jax.experimental.pallas import tpu as pltpu

N_DEV = 16
B, SQ, SKV, HQ_TOTAL, DH = 2, 512, 512, 128, 64
H_PER = HQ_TOTAL // N_DEV
D_MODEL = 768
ROWS = B * SQ

_HALVES = [512, 256, 128, 64]
_RBUF_OFFS = [0, 512, 768, 896]


def _fused_attn_allreduce(q, k, v, wo):

    def body(q_ref, k_ref, v_ref, wo_ref, out_ref, acc_ref, rbuf_ref,
             rs_send, rs_recv, ag_send, ag_recv):
        my = lax.axis_index("i")
        j = lax.rem(my, 4)
        z = lax.div(my, 4)
        bits = [
            jnp.where((j == 1) | (j == 2), 1, 0),
            jnp.where(j >= 2, 1, 0),
            lax.rem(z, 2),
            lax.div(z, 2),
        ]
        partners = [my ^ 1, my ^ 3, my ^ 4, my ^ 8]

        barrier_sem = pltpu.get_barrier_semaphore()
        for p in partners:
            pl.semaphore_signal(
                barrier_sem, inc=1,
                device_id=(p,), device_id_type=pl.DeviceIdType.MESH,
            )

        qi = lax.broadcasted_iota(jnp.int32, (SQ, SKV), 0)
        ki = lax.broadcasted_iota(jnp.int32, (SQ, SKV), 1)
        mask = (jnp.abs(qi - ki) <= 128) | (ki < 32) | (qi < 32)

        for b in range(B):
            for h in range(H_PER):
                qh = q_ref[b, h, :, :]
                kh = k_ref[b, h, :, :]
                vh = v_ref[b, h, :, :]
                s = lax.dot_general(
                    qh, kh, (((1,), (1,)), ((), ())),
                    preferred_element_type=jnp.float32,
                ) * 0.125
                s = jnp.where(mask, s, -1e9)
                m = jnp.max(s, axis=1, keepdims=True)
                e = jnp.exp(s - m)
                w = (e / jnp.sum(e, axis=1, keepdims=True)).astype(
                    jnp.bfloat16
                )
                ctx = lax.dot_general(
                    w, vh, (((1,), (0,)), ((), ())),
                    preferred_element_type=jnp.float32,
                ).astype(jnp.bfloat16)
                pw = lax.dot_general(
                    ctx, wo_ref[h * DH:(h + 1) * DH, :],
                    (((1,), (0,)), ((), ())),
                    preferred_element_type=jnp.float32,
                )
                if h == 0:
                    acc_ref[b * SQ:(b + 1) * SQ, :] = pw
                else:
                    acc_ref[b * SQ:(b + 1) * SQ, :] = (
                        acc_ref[b * SQ:(b + 1) * SQ, :] + pw
                    )

        out_ref[:, :] = acc_ref[:, :].astype(jnp.bfloat16)

        pl.semaphore_wait(barrier_sem, len(partners))

        S = jnp.int32(0)
        for k in range(4):
            h = _HALVES[k]
            send_off = S + (1 - bits[k]) * h
            keep_off = S + bits[k] * h
            rdma = pltpu.make_async_remote_copy(
                src_ref=out_ref.at[pl.ds(send_off, h)],
                dst_ref=rbuf_ref.at[pl.ds(_RBUF_OFFS[k], h)],
                send_sem=rs_send.at[k],
                recv_sem=rs_recv.at[k],
                device_id=(partners[k],),
                device_id_type=pl.DeviceIdType.MESH,
            )
            rdma.start()
            rdma.wait()
            out_ref[pl.ds(keep_off, h), :] = (
                out_ref[pl.ds(keep_off, h), :]
                + rbuf_ref[pl.ds(_RBUF_OFFS[k], h), :]
            )
            S = keep_off

        for k in (3, 2, 1, 0):
            g = _HALVES[k]
            rdma = pltpu.make_async_remote_copy(
                src_ref=out_ref.at[pl.ds(S, g)],
                dst_ref=out_ref.at[pl.ds(S, g)],
                send_sem=ag_send.at[k],
                recv_sem=ag_recv.at[k],
                device_id=(partners[k],),
                device_id_type=pl.DeviceIdType.MESH,
            )
            rdma.start()
            rdma.wait()
            S = S - bits[k] * g

    return pl.pallas_call(
        body,
        out_shape=jax.ShapeDtypeStruct((ROWS, D_MODEL), jnp.bfloat16),
        in_specs=[pl.BlockSpec(memory_space=pltpu.VMEM)] * 4,
        out_specs=pl.BlockSpec(memory_space=pltpu.VMEM),
        scratch_shapes=[
            pltpu.VMEM((ROWS, D_MODEL), jnp.float32),
            pltpu.VMEM((ROWS, D_MODEL), jnp.bfloat16),
            pltpu.SemaphoreType.DMA((4,)),
            pltpu.SemaphoreType.DMA((4,)),
            pltpu.SemaphoreType.DMA((4,)),
            pltpu.SemaphoreType.DMA((4,)),
        ],
        compiler_params=pltpu.CompilerParams(collective_id=0),
    )(q, k, v, wo)


def kernel(x, Wq, K_ext, V_ext, Wo):
    my = lax.axis_index("i")

    xb = x.astype(jnp.bfloat16)
    Q = jnp.einsum(
        "bsd,df->bsf", xb, Wq.astype(jnp.bfloat16),
        preferred_element_type=jnp.float32,
    ).reshape(B, SQ, H_PER, DH)
    Qt = jnp.transpose(Q, (0, 2, 1, 3)).astype(jnp.bfloat16)

    K = lax.dynamic_slice_in_dim(K_ext, my * H_PER, H_PER, axis=2)
    V = lax.dynamic_slice_in_dim(V_ext, my * H_PER, H_PER, axis=2)
    Kt = jnp.transpose(K, (0, 2, 1, 3)).astype(jnp.bfloat16)
    Vt = jnp.transpose(V, (0, 2, 1, 3)).astype(jnp.bfloat16)

    out = _fused_attn_allreduce(Qt, Kt, Vt, Wo.astype(jnp.bfloat16))
    return out.reshape(B, SQ, D_MODEL)
